# Initial kernel scaffold; baseline (speedup 1.0000x reference)
import jax
import jax.numpy as jnp
from jax.experimental import pallas as pl


def kernel(x, r_ij, r_ik, triple_ijk, neighbors_j, neighbors_k, triple_mask, W_in2f, W_filter, b_filter, W_f2out, b_f2out):
    raise NotImplementedError("write your pallas kernel here")


# speedup vs baseline: 27.1902x; 27.1902x over previous
import functools

import jax
import jax.numpy as jnp
from jax import lax
from jax.experimental import pallas as pl
from jax.experimental.pallas import tpu as pltpu
from jax.experimental.pallas import tpu_sc as plsc

B, At, Nbr = 8, 128, 512
NA = 25
AC = NA + 1
F = 128
NT = B * At
QW = AC * At
NC, NS, L = 2, 16, 16
TASKS_PER_W = NT // (NC * NS)


def _sc_hist_body(Tf, jf, kf, mf, R_out, Tv, jv, kv, mv, Qv):
    wid = lax.axis_index("s") * NC + lax.axis_index("c")
    lane = lax.iota(jnp.int32, L)
    l25 = lane * NA
    zero16 = jnp.zeros((L,), jnp.float32)

    def task_body(tt, carry):
        t = wid * TASKS_PER_W + tt
        pltpu.sync_copy(Tf.at[t], Tv)
        pltpu.sync_copy(jf.at[t], jv)
        pltpu.sync_copy(kf.at[t], kv)
        pltpu.sync_copy(mf.at[t], mv)

        def zbody(z, c):
            Qv[pl.ds(z * L, L)] = zero16
            return c

        lax.fori_loop(0, QW // L, zbody, 0)

        def gbody(g, c):
            jg = jv[pl.ds(g * L, L)]
            kg = kv[pl.ds(g * L, L)]
            mg = mv[pl.ds(g * L, L)]
            tbase = l25 + g * (L * NA)
            for a in range(NA):
                tvals = plsc.load_gather(Tv, [tbase + a])
                val = tvals * mg
                plsc.addupdate_scatter(Qv, [jg + (a * At)], val)
                plsc.addupdate_scatter(Qv, [kg + (a * At)], val)
            plsc.addupdate_scatter(Qv, [jg + (NA * At)], mg)
            plsc.addupdate_scatter(Qv, [kg + (NA * At)], mg)
            return c

        lax.fori_loop(0, Nbr // L, gbody, 0)
        pltpu.sync_copy(Qv, R_out.at[t])
        return carry

    lax.fori_loop(0, TASKS_PER_W, task_body, 0)


@jax.jit
def _sc_hist(Tf, jf, kf, mf):
    mesh = plsc.VectorSubcoreMesh(core_axis_name="c", subcore_axis_name="s")
    return pl.kernel(
        _sc_hist_body,
        out_type=jax.ShapeDtypeStruct((NT, QW), jnp.float32),
        mesh=mesh,
        scratch_types=[
            pltpu.VMEM((Nbr * NA,), jnp.float32),
            pltpu.VMEM((Nbr,), jnp.int32),
            pltpu.VMEM((Nbr,), jnp.int32),
            pltpu.VMEM((Nbr,), jnp.float32),
            pltpu.VMEM((QW,), jnp.float32),
        ],
        compiler_params=pltpu.CompilerParams(needs_layout_passes=False),
    )(Tf, jf, kf, mf)


def _tc_finish_body(x_ref, R_ref, Wi_ref, Wfe_ref, Wo_ref, bo_ref, out_ref, U2):
    y = jnp.dot(x_ref[0], Wi_ref[...], preferred_element_type=jnp.float32, precision=lax.Precision.HIGHEST)
    for a in range(AC):
        U2[a * At:(a + 1) * At, :] = y * Wfe_ref[a, :][None, :]
    yagg = jnp.dot(R_ref[0], U2[...], preferred_element_type=jnp.float32, precision=lax.Precision.HIGHEST)
    out_ref[0] = (
        jnp.dot(yagg, Wo_ref[...], preferred_element_type=jnp.float32, precision=lax.Precision.HIGHEST)
        + bo_ref[...]
    )


@jax.jit
def _tc_finish(x, R3, Wi, Wfe, Wo, bo2):
    return pl.pallas_call(
        _tc_finish_body,
        grid=(B,),
        in_specs=[
            pl.BlockSpec((1, At, F), lambda b: (b, 0, 0)),
            pl.BlockSpec((1, At, QW), lambda b: (b, 0, 0)),
            pl.BlockSpec((F, F), lambda b: (0, 0)),
            pl.BlockSpec((AC, F), lambda b: (0, 0)),
            pl.BlockSpec((F, F), lambda b: (0, 0)),
            pl.BlockSpec((1, F), lambda b: (0, 0)),
        ],
        out_specs=pl.BlockSpec((1, At, F), lambda b: (b, 0, 0)),
        out_shape=jax.ShapeDtypeStruct((B, At, F), jnp.float32),
        scratch_shapes=[pltpu.VMEM((QW, F), jnp.float32)],
    )(x, R3, Wi, Wfe, Wo, bo2)


def kernel(x, r_ij, r_ik, triple_ijk, neighbors_j, neighbors_k, triple_mask,
           W_in2f, W_filter, b_filter, W_f2out, b_f2out):
    Tf = triple_ijk.reshape(NT, Nbr * NA)
    jf = neighbors_j.reshape(NT, Nbr)
    kf = neighbors_k.reshape(NT, Nbr)
    mf = triple_mask.reshape(NT, Nbr)
    R = _sc_hist(Tf, jf, kf, mf)
    Wfe = jnp.concatenate([W_filter, b_filter[None, :]], axis=0)
    bo2 = b_f2out.reshape(1, F)
    return _tc_finish(x, R.reshape(B, At, QW), W_in2f, Wfe, W_f2out, bo2)


# speedup vs baseline: 27.2955x; 1.0039x over previous
import functools

import jax
import jax.numpy as jnp
from jax import lax
from jax.experimental import pallas as pl
from jax.experimental.pallas import tpu as pltpu
from jax.experimental.pallas import tpu_sc as plsc

B, At, Nbr = 8, 128, 512
NA = 25
AC = NA + 1
F = 128
NT = B * At
QW = AC * At
NC, NS, L = 2, 16, 16
TASKS_PER_W = NT // (NC * NS)


def _sc_hist_body(Tf, jf, kf, mf, R_out, Tv, jv, kv, mv, Qj, Qk):
    wid = lax.axis_index("s") * NC + lax.axis_index("c")
    lane = lax.iota(jnp.int32, L)
    l25 = lane * NA
    zero16 = jnp.zeros((L,), jnp.float32)

    def task_body(tt, carry):
        t = wid * TASKS_PER_W + tt
        pltpu.sync_copy(Tf.at[t], Tv)
        pltpu.sync_copy(jf.at[t], jv)
        pltpu.sync_copy(kf.at[t], kv)
        pltpu.sync_copy(mf.at[t], mv)

        @plsc.parallel_loop(0, QW // L, unroll=4)
        def _zero(z):
            Qj[pl.ds(z * L, L)] = zero16
            Qk[pl.ds(z * L, L)] = zero16

        @plsc.parallel_loop(0, Nbr // L, unroll=2)
        def _gather_scatter(g):
            jg = jv[pl.ds(g * L, L)]
            kg = kv[pl.ds(g * L, L)]
            mg = mv[pl.ds(g * L, L)]
            tbase = l25 + g * (L * NA)
            for a in range(NA):
                tvals = plsc.load_gather(Tv, [tbase + a])
                val = tvals * mg
                plsc.addupdate_scatter(Qj, [jg + (a * At)], val)
                plsc.addupdate_scatter(Qk, [kg + (a * At)], val)
            plsc.addupdate_scatter(Qj, [jg + (NA * At)], mg)
            plsc.addupdate_scatter(Qk, [kg + (NA * At)], mg)

        @plsc.parallel_loop(0, QW // L, unroll=4)
        def _merge(z):
            sl = pl.ds(z * L, L)
            Qj[sl] = Qj[sl] + Qk[sl]

        pltpu.sync_copy(Qj, R_out.at[t])
        return carry

    lax.fori_loop(0, TASKS_PER_W, task_body, 0)


@jax.jit
def _sc_hist(Tf, jf, kf, mf):
    mesh = plsc.VectorSubcoreMesh(core_axis_name="c", subcore_axis_name="s")
    return pl.kernel(
        _sc_hist_body,
        out_type=jax.ShapeDtypeStruct((NT, QW), jnp.float32),
        mesh=mesh,
        scratch_types=[
            pltpu.VMEM((Nbr * NA,), jnp.float32),
            pltpu.VMEM((Nbr,), jnp.int32),
            pltpu.VMEM((Nbr,), jnp.int32),
            pltpu.VMEM((Nbr,), jnp.float32),
            pltpu.VMEM((QW,), jnp.float32),
            pltpu.VMEM((QW,), jnp.float32),
        ],
        compiler_params=pltpu.CompilerParams(needs_layout_passes=False),
    )(Tf, jf, kf, mf)


def _tc_finish_body(x_ref, R_ref, Wi_ref, Wfe_ref, Wo_ref, bo_ref, out_ref, U2):
    y = jnp.dot(x_ref[0], Wi_ref[...], preferred_element_type=jnp.float32, precision=lax.Precision.HIGHEST)
    for a in range(AC):
        U2[a * At:(a + 1) * At, :] = y * Wfe_ref[a, :][None, :]
    yagg = jnp.dot(R_ref[0], U2[...], preferred_element_type=jnp.float32, precision=lax.Precision.HIGHEST)
    out_ref[0] = (
        jnp.dot(yagg, Wo_ref[...], preferred_element_type=jnp.float32, precision=lax.Precision.HIGHEST)
        + bo_ref[...]
    )


@jax.jit
def _tc_finish(x, R3, Wi, Wfe, Wo, bo2):
    return pl.pallas_call(
        _tc_finish_body,
        grid=(B,),
        in_specs=[
            pl.BlockSpec((1, At, F), lambda b: (b, 0, 0)),
            pl.BlockSpec((1, At, QW), lambda b: (b, 0, 0)),
            pl.BlockSpec((F, F), lambda b: (0, 0)),
            pl.BlockSpec((AC, F), lambda b: (0, 0)),
            pl.BlockSpec((F, F), lambda b: (0, 0)),
            pl.BlockSpec((1, F), lambda b: (0, 0)),
        ],
        out_specs=pl.BlockSpec((1, At, F), lambda b: (b, 0, 0)),
        out_shape=jax.ShapeDtypeStruct((B, At, F), jnp.float32),
        scratch_shapes=[pltpu.VMEM((QW, F), jnp.float32)],
    )(x, R3, Wi, Wfe, Wo, bo2)


def kernel(x, r_ij, r_ik, triple_ijk, neighbors_j, neighbors_k, triple_mask,
           W_in2f, W_filter, b_filter, W_f2out, b_f2out):
    Tf = triple_ijk.reshape(NT, Nbr * NA)
    jf = neighbors_j.reshape(NT, Nbr)
    kf = neighbors_k.reshape(NT, Nbr)
    mf = triple_mask.reshape(NT, Nbr)
    R = _sc_hist(Tf, jf, kf, mf)
    Wfe = jnp.concatenate([W_filter, b_filter[None, :]], axis=0)
    bo2 = b_f2out.reshape(1, F)
    return _tc_finish(x, R.reshape(B, At, QW), W_in2f, Wfe, W_f2out, bo2)


# speedup vs baseline: 34.7435x; 1.2729x over previous
import functools

import jax
import jax.numpy as jnp
from jax import lax
from jax.experimental import pallas as pl
from jax.experimental.pallas import tpu as pltpu
from jax.experimental.pallas import tpu_sc as plsc

B, At, Nbr = 8, 128, 512
NA = 25
AC = NA + 1
F = 128
NT = B * At
QW = AC * At
NC, NS, L = 2, 16, 16
TASKS_PER_W = NT // (NC * NS)


def _sc_hist_body(Tf, jf, kf, mf, R_out,
                  Tv0, jv0, kv0, mv0, Qj0, Qk0,
                  Tv1, jv1, kv1, mv1, Qj1, Qk1,
                  semi0, semi1, semo0, semo1):
    wid = lax.axis_index("s") * NC + lax.axis_index("c")
    lane = lax.iota(jnp.int32, L)
    l25 = lane * NA
    zero16 = jnp.zeros((L,), jnp.float32)
    t0 = wid * TASKS_PER_W
    set0 = (Tv0, jv0, kv0, mv0, Qj0, Qk0, semi0, semo0)
    set1 = (Tv1, jv1, kv1, mv1, Qj1, Qk1, semi1, semo1)

    def issue_in(t, bufs):
        Tv, jv, kv, mv, _, _, semi, _ = bufs
        pltpu.async_copy(Tf.at[t], Tv, semi)
        pltpu.async_copy(jf.at[t], jv, semi)
        pltpu.async_copy(kf.at[t], kv, semi)
        pltpu.async_copy(mf.at[t], mv, semi)

    def drain_in(bufs):
        Tv, jv, kv, mv, _, _, semi, _ = bufs
        pltpu.make_async_copy(Tf.at[t0], Tv, semi).wait()
        pltpu.make_async_copy(jf.at[t0], jv, semi).wait()
        pltpu.make_async_copy(kf.at[t0], kv, semi).wait()
        pltpu.make_async_copy(mf.at[t0], mv, semi).wait()

    def process(t, tt, bufs):
        Tv, jv, kv, mv, Qj, Qk, _, semo = bufs

        @pl.when(tt >= 2)
        def _():
            pltpu.make_async_copy(Qj, R_out.at[t0], semo).wait()

        @plsc.parallel_loop(0, QW // L, unroll=4)
        def _zero(z):
            Qj[pl.ds(z * L, L)] = zero16
            Qk[pl.ds(z * L, L)] = zero16

        @plsc.parallel_loop(0, Nbr // L, unroll=2)
        def _gather_scatter(g):
            jg = jv[pl.ds(g * L, L)]
            kg = kv[pl.ds(g * L, L)]
            mg = mv[pl.ds(g * L, L)]
            tbase = l25 + g * (L * NA)
            for a in range(NA):
                tvals = plsc.load_gather(Tv, [tbase + a])
                val = tvals * mg
                plsc.addupdate_scatter(Qj, [jg + (a * At)], val)
                plsc.addupdate_scatter(Qk, [kg + (a * At)], val)
            plsc.addupdate_scatter(Qj, [jg + (NA * At)], mg)
            plsc.addupdate_scatter(Qk, [kg + (NA * At)], mg)

        @plsc.parallel_loop(0, QW // L, unroll=4)
        def _merge(z):
            sl = pl.ds(z * L, L)
            Qj[sl] = Qj[sl] + Qk[sl]

        pltpu.async_copy(Qj, R_out.at[t], semo)

    issue_in(t0, set0)

    def pair_body(it, carry):
        tA = t0 + 2 * it
        issue_in(tA + 1, set1)
        drain_in(set0)
        process(tA, 2 * it, set0)

        @pl.when(it + 1 < TASKS_PER_W // 2)
        def _():
            issue_in(tA + 2, set0)

        drain_in(set1)
        process(tA + 1, 2 * it + 1, set1)
        return carry

    lax.fori_loop(0, TASKS_PER_W // 2, pair_body, 0)
    pltpu.make_async_copy(Qj0, R_out.at[t0], semo0).wait()
    pltpu.make_async_copy(Qj1, R_out.at[t0], semo1).wait()


@jax.jit
def _sc_hist(Tf, jf, kf, mf):
    mesh = plsc.VectorSubcoreMesh(core_axis_name="c", subcore_axis_name="s")
    return pl.kernel(
        _sc_hist_body,
        out_type=jax.ShapeDtypeStruct((NT, QW), jnp.float32),
        mesh=mesh,
        scratch_types=(
            [pltpu.VMEM((Nbr * NA,), jnp.float32),
             pltpu.VMEM((Nbr,), jnp.int32),
             pltpu.VMEM((Nbr,), jnp.int32),
             pltpu.VMEM((Nbr,), jnp.float32),
             pltpu.VMEM((QW,), jnp.float32),
             pltpu.VMEM((QW,), jnp.float32)] * 2
            + [pltpu.SemaphoreType.DMA] * 4
        ),
        compiler_params=pltpu.CompilerParams(needs_layout_passes=False),
    )(Tf, jf, kf, mf)


def _tc_finish_body(x_ref, R_ref, Wi_ref, Wfe_ref, Wo_ref, bo_ref, out_ref, U2):
    y = jnp.dot(x_ref[0], Wi_ref[...], preferred_element_type=jnp.float32, precision=lax.Precision.HIGHEST)
    for a in range(AC):
        U2[a * At:(a + 1) * At, :] = y * Wfe_ref[a, :][None, :]
    yagg = jnp.dot(R_ref[0], U2[...], preferred_element_type=jnp.float32, precision=lax.Precision.HIGHEST)
    out_ref[0] = (
        jnp.dot(yagg, Wo_ref[...], preferred_element_type=jnp.float32, precision=lax.Precision.HIGHEST)
        + bo_ref[...]
    )


@jax.jit
def _tc_finish(x, R3, Wi, Wfe, Wo, bo2):
    return pl.pallas_call(
        _tc_finish_body,
        grid=(B,),
        in_specs=[
            pl.BlockSpec((1, At, F), lambda b: (b, 0, 0)),
            pl.BlockSpec((1, At, QW), lambda b: (b, 0, 0)),
            pl.BlockSpec((F, F), lambda b: (0, 0)),
            pl.BlockSpec((AC, F), lambda b: (0, 0)),
            pl.BlockSpec((F, F), lambda b: (0, 0)),
            pl.BlockSpec((1, F), lambda b: (0, 0)),
        ],
        out_specs=pl.BlockSpec((1, At, F), lambda b: (b, 0, 0)),
        out_shape=jax.ShapeDtypeStruct((B, At, F), jnp.float32),
        scratch_shapes=[pltpu.VMEM((QW, F), jnp.float32)],
    )(x, R3, Wi, Wfe, Wo, bo2)


def kernel(x, r_ij, r_ik, triple_ijk, neighbors_j, neighbors_k, triple_mask,
           W_in2f, W_filter, b_filter, W_f2out, b_f2out):
    Tf = triple_ijk.reshape(NT, Nbr * NA)
    jf = neighbors_j.reshape(NT, Nbr)
    kf = neighbors_k.reshape(NT, Nbr)
    mf = triple_mask.reshape(NT, Nbr)
    R = _sc_hist(Tf, jf, kf, mf)
    Wfe = jnp.concatenate([W_filter, b_filter[None, :]], axis=0)
    bo2 = b_f2out.reshape(1, F)
    return _tc_finish(x, R.reshape(B, At, QW), W_in2f, Wfe, W_f2out, bo2)
